# baseline (speedup 1.0000x reference)
import functools

import jax
import jax.numpy as jnp
from jax import lax
from jax.experimental import pallas as pl
from jax.experimental.pallas import tpu as pltpu
from jax.experimental.pallas import tpu_sc as plsc

D_MODEL = 64
NB = 512
SEQ = 50
BATCH = 16384
SCALE = 64.0
LANES = 16
NGBUF = 3


@jax.jit
def _run(xidx, lut):
    info = plsc.get_sparse_core_info()
    nc = info.num_cores
    n_workers = nc * info.num_subcores
    chunks_per_s = BATCH // NB
    total_chunks = SEQ * chunks_per_s
    cpw = total_chunks // n_workers
    mesh = plsc.VectorSubcoreMesh(core_axis_name="c", subcore_axis_name="s")

    @functools.partial(
        pl.kernel,
        mesh=mesh,
        out_type=jax.ShapeDtypeStruct((SEQ, BATCH, D_MODEL), jnp.float32),
        scratch_types=[
            pltpu.VMEM((cpw, NB // 128, 128), jnp.int32),
            pltpu.VMEM((NGBUF, NB, D_MODEL), jnp.float32),
            pltpu.SemaphoreType.DMA((NGBUF,)),
            pltpu.SemaphoreType.DMA((NGBUF,)),
        ],
        compiler_params=pltpu.CompilerParams(
            use_tc_tiling_on_sc=False, needs_layout_passes=False
        ),
    )
    def k(xidx_hbm, lut_hbm, out_hbm, idx_v, gbuf, gsem, osem):
        wid = lax.axis_index("s") * nc + lax.axis_index("c")
        c0 = wid * cpw
        pltpu.sync_copy(xidx_hbm.at[pl.ds(c0, cpw)], idx_v)

        def start_gathers(ci, b):
            for g in range(NB // 128):
                pltpu.async_copy(
                    lut_hbm.at[idx_v.at[ci, g]],
                    gbuf.at[b, pl.ds(g * 128, 128)],
                    gsem.at[b],
                )

        def wait_gathers(ci, b):
            for g in range(NB // 128):
                pltpu.make_async_copy(
                    lut_hbm.at[idx_v.at[ci, g]],
                    gbuf.at[b, pl.ds(g * 128, 128)],
                    gsem.at[b],
                ).wait()

        def out_slice(ci):
            c = c0 + ci
            s = c // chunks_per_s
            b0 = (c % chunks_per_s) * NB
            return out_hbm.at[s, pl.ds(b0, NB)]

        def wait_out(ci, b):
            pltpu.make_async_copy(gbuf.at[b], out_slice(ci), osem.at[b]).wait()

        for b in range(NGBUF):
            start_gathers(b, b)

        def step(ci, carry):
            gb = lax.rem(ci, NGBUF)
            wait_gathers(ci, gb)

            def srow(r4, carry2):
                r0 = r4 * 4
                for rr in range(4):
                    r = r0 + rr
                    for g in range(D_MODEL // LANES):
                        sl = pl.ds(g * LANES, LANES)
                        gbuf[gb, r, sl] = gbuf[gb, r, sl] * SCALE
                return carry2

            lax.fori_loop(0, NB // 4, srow, 0)
            pltpu.async_copy(gbuf.at[gb], out_slice(ci), osem.at[gb])

            @pl.when(ci < cpw - NGBUF)
            def _refill():
                wait_out(ci, gb)
                start_gathers(ci + NGBUF, gb)

            return carry

        lax.fori_loop(0, cpw, step, 0)
        for b in range(NGBUF):
            ci = cpw - NGBUF + b
            wait_out(ci, lax.rem(ci, NGBUF))

    return k(xidx, lut)


def kernel(x, lut):
    xidx = jnp.transpose(x).astype(jnp.int32).reshape(SEQ * BATCH // NB, NB // 128, 128)
    o = _run(xidx, lut)
    return jnp.transpose(o, (1, 0, 2))
